# baseline (speedup 1.0000x reference)
import jax
import jax.numpy as jnp
import numpy as np
from jax.experimental import pallas as pl
import jax.experimental.pallas.tpu as pltpu

N_EMBED = 1024
E = 8
EP = 128
TOP_K = 2
B_ = 4
S_ = 8192
BS = 2048
NS = S_ // BS

_NOISE = np.array([
    [-0.02830461598932743, 0.4671318531036377, 0.2957029640674591,
     0.15354591608047485, -0.12403281778097153, 0.21692314743995667,
     -1.440878987312317, 0.755859911441803],
    [0.5214096307754517, 0.9101703763008118, -0.3844965994358063,
     1.139823317527771, 1.4457862377166748, 1.080906629562378,
     -0.05629321187734604, 0.9095944762229919],
    [0.5573461651802063, 0.21905718743801117, -1.4485087394714355,
     0.7641875147819519, -0.24154697358608246, -1.179381012916565,
     -1.9389183521270752, 0.3562646210193634],
    [-0.24111966788768768, 1.2151274681091309, -1.3952220678329468,
     -0.5347688794136047, 0.27067556977272034, 1.5401241779327393,
     0.6935186386108398, -0.1038767620921135],
], dtype=np.float32)

def _router_kernel(x_ref, avg_ref, wr_ref, br_ref, wn_ref, bn_ref,
                   router_ref, idx_ref, acc_ref):
    b = pl.program_id(0)
    s = pl.program_id(1)

    @pl.when((b == 0) & (s == 0))
    def _init():
        acc_ref[...] = jnp.zeros_like(acc_ref)

    x = x_ref[0]
    yt = jax.lax.dot_general(wn_ref[...], x,
                             (((1,), (1,)), ((), ())),
                             preferred_element_type=jnp.float32)
    yt = yt + bn_ref[...].reshape(E, 1)
    sp = jnp.maximum(yt, 0.0) + jnp.log1p(jnp.exp(-jnp.abs(yt)))
    part = jnp.sum(sp, axis=1, keepdims=True)
    lane = jax.lax.broadcasted_iota(jnp.int32, (E, EP), 1)
    acc_ref[...] += jnp.where(lane == b, part, 0.0)

    @pl.when((b == B_ - 1) & (s == NS - 1))
    def _epilogue():
        mean = jnp.transpose(acc_ref[...])[:B_, :E] * (1.0 / S_)
        logits = jax.lax.dot_general(avg_ref[...], wr_ref[...],
                                     (((1,), (1,)), ((), ())),
                                     preferred_element_type=jnp.float32)
        col = jax.lax.broadcasted_iota(jnp.int32, (B_, E), 1)
        row = jax.lax.broadcasted_iota(jnp.int32, (B_, E), 0)
        ns = jnp.zeros((B_, E), jnp.float32)
        for bb in range(B_):
            for ee in range(E):
                ns = jnp.where((row == bb) & (col == ee),
                               float(_NOISE[bb, ee]), ns)
        noisy = (logits + br_ref[...].reshape(1, E)
                 + ns * mean)
        neg = jnp.float32(-1e30)
        m1 = jnp.max(noisy, axis=1, keepdims=True)
        i1 = jnp.min(jnp.where(noisy == m1, col, E), axis=1, keepdims=True)
        rest = jnp.where(col == i1, neg, noisy)
        m2 = jnp.max(rest, axis=1, keepdims=True)
        i2 = jnp.min(jnp.where(rest == m2, col, E), axis=1, keepdims=True)
        d = jnp.exp(m2 - m1)
        p1 = 1.0 / (1.0 + d)
        p2 = d / (1.0 + d)
        router_ref[...] = jnp.where(col == i1, p1,
                                    jnp.where(col == i2, p2, 0.0))
        idx_ref[:, 0:1] = i1
        idx_ref[:, 1:2] = i2


def kernel(mh_output, mh_output_avg, W_route, b_route, W_noise, b_noise):
    return pl.pallas_call(
        _router_kernel,
        grid=(B_, NS),
        in_specs=[
            pl.BlockSpec((1, BS, N_EMBED), lambda b, s: (b, s, 0)),
            pl.BlockSpec((B_, N_EMBED), lambda b, s: (0, 0)),
            pl.BlockSpec((E, N_EMBED), lambda b, s: (0, 0)),
            pl.BlockSpec((E,), lambda b, s: (0,)),
            pl.BlockSpec((E, N_EMBED), lambda b, s: (0, 0)),
            pl.BlockSpec((E,), lambda b, s: (0,)),
        ],
        out_specs=[
            pl.BlockSpec((B_, E), lambda b, s: (0, 0)),
            pl.BlockSpec((B_, TOP_K), lambda b, s: (0, 0)),
        ],
        out_shape=[
            jax.ShapeDtypeStruct((B_, E), jnp.float32),
            jax.ShapeDtypeStruct((B_, TOP_K), jnp.int32),
        ],
        scratch_shapes=[pltpu.VMEM((E, EP), jnp.float32)],
    )(mh_output, mh_output_avg, W_route, b_route, W_noise, b_noise)
